# baseline (speedup 1.0000x reference)
import functools

import jax
import jax.numpy as jnp
from jax import lax
from jax.experimental import pallas as pl
from jax.experimental.pallas import tpu as pltpu
from jax.experimental.pallas import tpu_sc as plsc

_F = 128
_K = 65536
_NC = 2
_NS = 16
_NW = _NC * _NS
_CPW = _K // _NW
_C = 256
_W = 264
_NCHUNK = _CPW // _C

_mesh = plsc.VectorSubcoreMesh(core_axis_name="c", subcore_axis_name="s")


@functools.partial(
    pl.kernel,
    out_type=jax.ShapeDtypeStruct((_K, _F), jnp.float32),
    mesh=_mesh,
    scratch_types=[
        pltpu.VMEM((_F, _W), jnp.float32),
        pltpu.VMEM((_F, _W), jnp.float32),
        pltpu.VMEM((_C, _F), jnp.float32),
        pltpu.SemaphoreType.DMA,
        pltpu.SemaphoreType.DMA,
        pltpu.SemaphoreType.DMA,
    ],
    compiler_params=pltpu.CompilerParams(
        needs_layout_passes=False, use_tc_tiling_on_sc=False),
)
def _sc_transpose(q_hbm, out_hbm, in_a, in_b, out_v, sem_ia, sem_ib, sem_o):
    wid = lax.axis_index("s") * _NC + lax.axis_index("c")
    col0 = wid * _CPW
    iota = lax.iota(jnp.int32, 16)
    frows = [iota + f0 for f0 in range(0, _F, 16)]

    def _read_start(ch):
        c0 = col0 + ch * _C
        return jnp.where(c0 + _W > _K, c0 - 8, c0)

    def _in_slice(ch):
        return q_hbm.at[:, pl.ds(_read_start(ch), _W)]

    def _out_slice(ch):
        return out_hbm.at[pl.ds(col0 + ch * _C, _C), :]

    def _permute(in_v, out_v_, delta):
        @plsc.parallel_loop(0, _C, unroll=2)
        def _row(k):
            cols = jnp.full((16,), k + delta, jnp.int32)
            for j in range(_F // 16):
                v = plsc.load_gather(in_v, [frows[j], cols])
                out_v_[k, pl.ds(j * 16, 16)] = v

    def _half(ch, in_v, sem_i):
        pltpu.make_async_copy(_in_slice(ch), in_v, sem_i).wait()

        @pl.when(ch > 0)
        def _():
            pltpu.make_async_copy(out_v, _out_slice(ch), sem_o).wait()

        c0 = col0 + ch * _C
        delta = c0 - _read_start(ch)
        _permute(in_v, out_v, delta)
        pltpu.async_copy(out_v, _out_slice(ch), sem_o)

        @pl.when(ch + 2 < _NCHUNK)
        def _():
            pltpu.async_copy(_in_slice(ch + 2), in_v, sem_i)

    pltpu.async_copy(_in_slice(0), in_a, sem_ia)
    pltpu.async_copy(_in_slice(1), in_b, sem_ib)

    def _pair(p, carry):
        _half(2 * p, in_a, sem_ia)
        _half(2 * p + 1, in_b, sem_ib)
        return carry

    lax.fori_loop(0, _NCHUNK // 2, _pair, 0)
    pltpu.make_async_copy(out_v, _out_slice(_NCHUNK - 1), sem_o).wait()


def kernel(queue):
    return _sc_transpose(queue)
